# baseline (speedup 1.0000x reference)
import functools

import jax
import jax.numpy as jnp
from jax import lax
from jax.experimental import pallas as pl
from jax.experimental.pallas import tpu as pltpu
from jax.experimental.pallas import tpu_sc as plsc

_NC, _NS, _L = 2, 16, 16
_NW = _NC * _NS
_NB = 4


@functools.lru_cache(maxsize=None)
def _make_sc_embed(N, S, V, C, D):
    n_per_w = N // _NW
    blk = n_per_w // _NB
    mesh = plsc.VectorSubcoreMesh(
        core_axis_name="c", subcore_axis_name="s",
        num_cores=_NC, num_subcores=_NS,
    )

    @functools.partial(
        pl.kernel,
        out_type=jax.ShapeDtypeStruct((N, D), jnp.float32),
        mesh=mesh,
        scratch_types=[
            pltpu.VMEM((_NB, blk), jnp.int32),
            pltpu.VMEM((n_per_w, D), jnp.float32),
            pltpu.VMEM((n_per_w, D), jnp.float32),
            pltpu.SemaphoreType.DMA((_NB,)),
            pltpu.SemaphoreType.DMA,
        ],
    )
    def embed(idx_hbm, tok_hbm, pos_hbm, out_hbm, idx_v, rows_v, pos_v,
              gsem, ssem):
        wid = lax.axis_index("s") * _NC + lax.axis_index("c")
        base = wid * n_per_w
        p0 = lax.rem(base, S)

        for j in range(_NB):
            pltpu.sync_copy(idx_hbm.at[pl.ds(base + j * blk, blk)],
                            idx_v.at[j])
        gathers = [
            pltpu.async_copy(tok_hbm.at[idx_v.at[j]],
                             rows_v.at[pl.ds(j * blk, blk)], gsem.at[j])
            for j in range(_NB)
        ]
        pltpu.sync_copy(pos_hbm.at[pl.ds(p0, n_per_w)], pos_v)

        stores = []
        for j in range(_NB):
            gathers[j].wait()

            @plsc.parallel_loop(j * blk, (j + 1) * blk, unroll=4)
            def _add_pos(r):
                for c in range(D // _L):
                    sl = pl.ds(c * _L, _L)
                    rows_v[r, sl] = rows_v[r, sl] + pos_v[r, sl]

            stores.append(
                pltpu.async_copy(rows_v.at[pl.ds(j * blk, blk)],
                                 out_hbm.at[pl.ds(base + j * blk, blk)],
                                 ssem))
        for s in stores:
            s.wait()

    return embed


def kernel(in_idx, tok_emb, pos_emb):
    B, S = in_idx.shape
    V, D = tok_emb.shape
    C = pos_emb.shape[0]
    N = B * S
    idx = in_idx.reshape(N).astype(jnp.int32)
    out = _make_sc_embed(N, S, V, C, D)(idx, tok_emb, pos_emb)
    return out.reshape(B, S, D)


# speedup vs baseline: 1.1039x; 1.1039x over previous
import functools

import jax
import jax.numpy as jnp
from jax import lax
from jax.experimental import pallas as pl
from jax.experimental.pallas import tpu as pltpu
from jax.experimental.pallas import tpu_sc as plsc

_NC, _NS, _L = 2, 16, 16
_NW = _NC * _NS
_CHUNK = 128


@functools.lru_cache(maxsize=None)
def _make_sc_embed(N, S, V, C, D):
    n_per_w = N // _NW
    n_chunks = n_per_w // _CHUNK
    mesh = plsc.VectorSubcoreMesh(
        core_axis_name="c", subcore_axis_name="s",
        num_cores=_NC, num_subcores=_NS,
    )

    @functools.partial(
        pl.kernel,
        out_type=jax.ShapeDtypeStruct((N, D), jnp.float32),
        mesh=mesh,
        scratch_types=[
            pltpu.VMEM((n_chunks, _CHUNK), jnp.int32),
            pltpu.VMEM((n_per_w, D), jnp.float32),
            pltpu.SemaphoreType.DMA((n_chunks,)),
        ],
    )
    def embed(idx_hbm, tok_hbm, pos_hbm, out_hbm, idx_v, rows_v, gsem):
        wid = lax.axis_index("s") * _NC + lax.axis_index("c")
        base = wid * n_per_w
        p0 = lax.rem(base, S)

        for j in range(n_chunks):
            pltpu.sync_copy(idx_hbm.at[pl.ds(base + j * _CHUNK, _CHUNK)],
                            idx_v.at[j])
        pltpu.sync_copy(pos_hbm.at[pl.ds(p0, n_per_w)], rows_v)
        gathers = [
            pltpu.async_copy(tok_hbm.at[idx_v.at[j]],
                             rows_v.at[pl.ds(j * _CHUNK, _CHUNK)],
                             gsem.at[j], add=True)
            for j in range(n_chunks)
        ]
        for g in gathers:
            g.wait()
        pltpu.sync_copy(rows_v, out_hbm.at[pl.ds(base, n_per_w)])

    return embed


def kernel(in_idx, tok_emb, pos_emb):
    B, S = in_idx.shape
    V, D = tok_emb.shape
    C = pos_emb.shape[0]
    N = B * S
    idx = in_idx.reshape(N).astype(jnp.int32)
    out = _make_sc_embed(N, S, V, C, D)(idx, tok_emb, pos_emb)
    return out.reshape(B, S, D)
